# baseline (speedup 1.0000x reference)
import functools

import jax
import jax.numpy as jnp
from jax import lax
from jax.experimental import pallas as pl
from jax.experimental.pallas import tpu as pltpu
from jax.experimental.pallas import tpu_sc as plsc

_VOCAB = 1000000
_EMBED_DIM = 32
_HALF = _EMBED_DIM // 2
_BATCH = 16384
_FIELDS = 26
_NW = 32
_BSL = _BATCH // _NW
_NBUF = 2

_mesh = plsc.VectorSubcoreMesh(core_axis_name="c", subcore_axis_name="s")


@functools.partial(
    pl.kernel,
    out_type=jax.ShapeDtypeStruct((_FIELDS, _BATCH, _EMBED_DIM), jnp.float32),
    mesh=_mesh,
    scratch_types=[
        pltpu.VMEM((_FIELDS * _BSL,), jnp.int32),
        pltpu.VMEM((_NBUF, _BSL, _HALF), jnp.float32),
        pltpu.VMEM((_NBUF, _BSL, _HALF), jnp.float32),
        pltpu.SemaphoreType.DMA,
        [pltpu.SemaphoreType.DMA] * _NBUF,
        [pltpu.SemaphoreType.DMA] * _NBUF,
        [pltpu.SemaphoreType.DMA] * _NBUF,
        [pltpu.SemaphoreType.DMA] * _NBUF,
    ],
    compiler_params=pltpu.CompilerParams(use_tc_tiling_on_sc=False),
)
def _gather(tab_lo, tab_hi, idx_hbm, out_hbm, idx_fm, rows_lo, rows_hi,
            sem_ix, sems_glo, sems_ghi, sems_wlo, sems_whi):
    wid = lax.axis_index("s") * 2 + lax.axis_index("c")
    b0 = wid * _BSL

    for f in range(_FIELDS):
        pltpu.async_copy(
            idx_hbm.at[f, pl.ds(b0, _BSL)], idx_fm.at[pl.ds(f * _BSL, _BSL)], sem_ix
        )
    for f in range(_FIELDS):
        pltpu.make_async_copy(
            idx_hbm.at[0, pl.ds(0, _BSL)], idx_fm.at[pl.ds(0, _BSL)], sem_ix
        ).wait()

    def gather_start(f, b):
        idx_chunk = idx_fm.at[pl.ds(f * _BSL, _BSL)]
        return (
            pltpu.async_copy(tab_lo.at[idx_chunk], rows_lo.at[b], sems_glo[b]),
            pltpu.async_copy(tab_hi.at[idx_chunk], rows_hi.at[b], sems_ghi[b]),
        )

    def write_start(f, b):
        return (
            pltpu.async_copy(
                rows_lo.at[b], out_hbm.at[f, pl.ds(b0, _BSL), pl.ds(0, _HALF)],
                sems_wlo[b],
            ),
            pltpu.async_copy(
                rows_hi.at[b], out_hbm.at[f, pl.ds(b0, _BSL), pl.ds(_HALF, _HALF)],
                sems_whi[b],
            ),
        )

    gathers = [gather_start(b, b) for b in range(_NBUF)]
    writes = [None] * _NBUF

    for f in range(_FIELDS):
        b = f % _NBUF
        for g in gathers[b]:
            g.wait()
        writes[b] = write_start(f, b)
        nxt = f + _NBUF
        if nxt < _FIELDS:
            for w in writes[b]:
                w.wait()
            gathers[b] = gather_start(nxt, b)

    for b in range(_NBUF):
        if writes[b] is not None:
            for w in writes[b]:
                w.wait()


def kernel(embedding, indices):
    tab_lo = embedding[:, :_HALF]
    tab_hi = embedding[:, _HALF:]
    out = _gather(tab_lo, tab_hi, indices.T)
    return out.transpose(1, 0, 2)


# speedup vs baseline: 1.9716x; 1.9716x over previous
import functools

import jax
import jax.numpy as jnp
from jax import lax
from jax.experimental import pallas as pl
from jax.experimental.pallas import tpu as pltpu
from jax.experimental.pallas import tpu_sc as plsc

_VOCAB = 1000000
_EMBED_DIM = 32
_BATCH = 16384
_FIELDS = 26
_NW = 32
_BSL = _BATCH // _NW
_NBUF = 2

_mesh = plsc.VectorSubcoreMesh(core_axis_name="c", subcore_axis_name="s")


@functools.partial(
    pl.kernel,
    out_type=jax.ShapeDtypeStruct((_FIELDS, _BATCH, _EMBED_DIM), jnp.float32),
    mesh=_mesh,
    scratch_types=[
        pltpu.VMEM((_FIELDS * _BSL,), jnp.int32),
        pltpu.VMEM((_NBUF, _BSL, _EMBED_DIM), jnp.float32),
        pltpu.SemaphoreType.DMA,
        [pltpu.SemaphoreType.DMA] * _NBUF,
        [pltpu.SemaphoreType.DMA] * _NBUF,
    ],
    compiler_params=pltpu.CompilerParams(use_tc_tiling_on_sc=False),
)
def _gather(table_hbm, idx_hbm, out_hbm, idx_fm, rows_v, sem_ix, sems_g, sems_w):
    wid = lax.axis_index("s") * 2 + lax.axis_index("c")
    b0 = wid * _BSL

    for f in range(_FIELDS):
        pltpu.async_copy(
            idx_hbm.at[f, pl.ds(b0, _BSL)], idx_fm.at[pl.ds(f * _BSL, _BSL)], sem_ix
        )
    for f in range(_FIELDS):
        pltpu.make_async_copy(
            idx_hbm.at[0, pl.ds(0, _BSL)], idx_fm.at[pl.ds(0, _BSL)], sem_ix
        ).wait()

    def gather_start(f, b):
        idx_chunk = idx_fm.at[pl.ds(f * _BSL, _BSL)]
        return pltpu.async_copy(table_hbm.at[idx_chunk], rows_v.at[b], sems_g[b])

    def write_start(f, b):
        dst = out_hbm.at[f, pl.ds(b0, _BSL), :]
        return pltpu.async_copy(rows_v.at[b], dst, sems_w[b])

    gathers = [gather_start(b, b) for b in range(_NBUF)]
    writes = [None] * _NBUF

    for f in range(_FIELDS):
        b = f % _NBUF
        gathers[b].wait()
        writes[b] = write_start(f, b)
        nxt = f + _NBUF
        if nxt < _FIELDS:
            writes[b].wait()
            gathers[b] = gather_start(nxt, b)

    for b in range(_NBUF):
        if writes[b] is not None:
            writes[b].wait()


def kernel(embedding, indices):
    out = _gather(embedding, indices.T)
    return out.transpose(1, 0, 2)
